# Initial kernel scaffold; baseline (speedup 1.0000x reference)
import jax
import jax.numpy as jnp
from jax.experimental import pallas as pl


def kernel(x, edge_index, W1l, W1r, b1, W2l, W2r, b2):
    raise NotImplementedError("write your pallas kernel here")


# speedup vs baseline: 6.7211x; 6.7211x over previous
import functools

import jax
import jax.numpy as jnp
from jax import lax
from jax.experimental import pallas as pl
from jax.experimental.pallas import tpu as pltpu
from jax.experimental.pallas import tpu_sc as plsc

FIN = 128
FH = 64
N_NODES = 10000
N_EDGES = 320000

NC, NS, L = 2, 16, 16
CW = 128
NCHUNK = 158
EDGES_PER_W = NCHUNK * CW
E_PAD = NS * EDGES_PER_W
N_ACC = 10240
ROWS_PER_TILE = N_ACC // NS
CNTW = 8
BR = 1024


def _sc_agg_body(with_counts, *refs):
    if with_counts:
        (feat, srcp, dstp, zrow, zcnt8, ones8, out, cnt_out,
         src_v, dst_v, buf0, buf1, ones_v, acc_sh, cnt_sh, sem0, sem1) = refs
    else:
        (feat, srcp, dstp, zrow, out,
         src_v, dst_v, buf0, buf1, acc_sh, sem0, sem1) = refs

    c = lax.axis_index("c")
    s = lax.axis_index("s")
    stripe = pl.ds(s * ROWS_PER_TILE, ROWS_PER_TILE)

    pltpu.sync_copy(zrow, acc_sh.at[stripe])
    if with_counts:
        pltpu.sync_copy(ones8, ones_v)

        @pl.when(c == 0)
        def _():
            pltpu.sync_copy(zcnt8, cnt_sh.at[stripe])

    plsc.subcore_barrier()

    pltpu.sync_copy(srcp.at[s], src_v)
    pltpu.sync_copy(dstp.at[s], dst_v)

    half = feat.at[c]
    pltpu.async_copy(half.at[src_v.at[0]], buf0, sem0)
    pltpu.async_copy(half.at[src_v.at[1]], buf1, sem1)

    def body(i, carry):
        j0 = 2 * i
        pltpu.make_async_copy(half.at[src_v.at[j0]], buf0, sem0).wait()
        pltpu.sync_copy(buf0, acc_sh.at[dst_v.at[j0]], add=True)
        if with_counts:
            @pl.when(c == 0)
            def _():
                pltpu.sync_copy(ones_v, cnt_sh.at[dst_v.at[j0]], add=True)

        @pl.when(j0 + 2 < NCHUNK)
        def _():
            pltpu.async_copy(half.at[src_v.at[j0 + 2]], buf0, sem0)

        pltpu.make_async_copy(half.at[src_v.at[j0 + 1]], buf1, sem1).wait()
        pltpu.sync_copy(buf1, acc_sh.at[dst_v.at[j0 + 1]], add=True)
        if with_counts:
            @pl.when(c == 0)
            def _():
                pltpu.sync_copy(ones_v, cnt_sh.at[dst_v.at[j0 + 1]], add=True)

        @pl.when(j0 + 3 < NCHUNK)
        def _():
            pltpu.async_copy(half.at[src_v.at[j0 + 3]], buf1, sem1)

        return carry

    lax.fori_loop(0, NCHUNK // 2, body, 0)

    plsc.subcore_barrier()
    pltpu.sync_copy(acc_sh.at[stripe], out.at[c, stripe])
    if with_counts:
        @pl.when(c == 0)
        def _():
            pltpu.sync_copy(cnt_sh.at[stripe], cnt_out.at[stripe])


def _make_sc_agg(with_counts):
    out_type = [jax.ShapeDtypeStruct((NC, N_ACC, FH), jnp.float32)]
    if with_counts:
        out_type.append(jax.ShapeDtypeStruct((N_ACC, CNTW), jnp.float32))
    scratch = [
        pltpu.VMEM((NCHUNK, CW), jnp.int32),
        pltpu.VMEM((NCHUNK, CW), jnp.int32),
        pltpu.VMEM((CW, FH), jnp.float32),
        pltpu.VMEM((CW, FH), jnp.float32),
    ]
    if with_counts:
        scratch.append(pltpu.VMEM((CW, CNTW), jnp.float32))
    scratch.append(pltpu.VMEM_SHARED((N_ACC, FH), jnp.float32))
    if with_counts:
        scratch.append(pltpu.VMEM_SHARED((N_ACC, CNTW), jnp.float32))
    scratch += [
        pltpu.SemaphoreType.DMA,
        pltpu.SemaphoreType.DMA,
    ]
    return pl.kernel(
        functools.partial(_sc_agg_body, with_counts),
        out_type=out_type,
        mesh=plsc.VectorSubcoreMesh(core_axis_name="c", subcore_axis_name="s"),
        scratch_types=scratch,
        compiler_params=pltpu.CompilerParams(use_tc_tiling_on_sc=False),
    )


def _tc_layer_body(relu, split_out, sp_ref, cnt_ref, x_ref, wl_ref, wr_ref,
                   b_ref, o_ref):
    cnt = cnt_ref[...][:, 0]
    inv = 1.0 / jnp.maximum(cnt, 1.0)
    wl = wl_ref[...]
    wr = wr_ref[...]
    y = (jnp.dot(sp_ref[0] * inv[:, None], wl[:FH],
                 preferred_element_type=jnp.float32)
         + jnp.dot(sp_ref[1] * inv[:, None], wl[FH:],
                   preferred_element_type=jnp.float32)
         + jnp.dot(x_ref[0], wr[:FH], preferred_element_type=jnp.float32)
         + jnp.dot(x_ref[1], wr[FH:], preferred_element_type=jnp.float32)
         + b_ref[...])
    if relu:
        y = jnp.maximum(y, 0.0)
    if split_out:
        o_ref[0] = y[:, :FH]
        o_ref[1] = y[:, FH:]
    else:
        o_ref[...] = y


def _tc_layer(relu, split_out, sp, cnt, x2, wl, wr, b):
    if split_out:
        out_shape = jax.ShapeDtypeStruct((NC, N_ACC, FH), jnp.float32)
        out_spec = pl.BlockSpec((NC, BR, FH), lambda i: (0, i, 0))
    else:
        out_shape = jax.ShapeDtypeStruct((N_ACC, FIN), jnp.float32)
        out_spec = pl.BlockSpec((BR, FIN), lambda i: (i, 0))
    return pl.pallas_call(
        functools.partial(_tc_layer_body, relu, split_out),
        grid=(N_ACC // BR,),
        in_specs=[
            pl.BlockSpec((NC, BR, FH), lambda i: (0, i, 0)),
            pl.BlockSpec((BR, CNTW), lambda i: (i, 0)),
            pl.BlockSpec((NC, BR, FH), lambda i: (0, i, 0)),
            pl.BlockSpec((FIN, FIN), lambda i: (0, 0)),
            pl.BlockSpec((FIN, FIN), lambda i: (0, 0)),
            pl.BlockSpec((1, FIN), lambda i: (0, 0)),
        ],
        out_specs=out_spec,
        out_shape=out_shape,
    )(sp, cnt, x2, wl, wr, b)


def kernel(x, edge_index, W1l, W1r, b1, W2l, W2r, b2):
    src = edge_index[0].astype(jnp.int32)
    dst = edge_index[1].astype(jnp.int32)
    npad = E_PAD - N_EDGES
    srcp = jnp.concatenate([src, jnp.zeros((npad,), jnp.int32)])
    dstp = jnp.concatenate([dst, jnp.full((npad,), N_NODES, jnp.int32)])
    srcp = srcp.reshape(NS, NCHUNK, CW)
    dstp = dstp.reshape(NS, NCHUNK, CW)
    xp = jnp.pad(x, ((0, N_ACC - N_NODES), (0, 0)))
    x2 = jnp.stack([xp[:, :FH], xp[:, FH:]])
    zrow = jnp.zeros((ROWS_PER_TILE, FH), jnp.float32)
    zcnt8 = jnp.zeros((ROWS_PER_TILE, CNTW), jnp.float32)
    ones8 = jnp.ones((CW, CNTW), jnp.float32)

    s1, cnt = _make_sc_agg(True)(x2, srcp, dstp, zrow, zcnt8, ones8)
    h2 = _tc_layer(True, True, s1, cnt, x2, W1l, W1r, b1.reshape(1, FIN))
    (s2,) = _make_sc_agg(False)(h2, srcp, dstp, zrow)
    out = _tc_layer(False, False, s2, cnt, h2, W2l, W2r, b2.reshape(1, FIN))
    return out[:N_NODES]
